# baseline (speedup 1.0000x reference)
import jax
import jax.numpy as jnp
from jax.experimental import pallas as pl

_BM = 256
_BN = 6144


def _matmul_sigmoid_kernel(u_ref, v_ref, out_ref):
    acc = jax.lax.dot_general(
        u_ref[...] * 0.5,
        v_ref[...],
        dimension_numbers=(((1,), (1,)), ((), ())),
        preferred_element_type=jnp.float32,
    )
    out_ref[...] = 0.5 * jnp.tanh(acc) + 0.5


def kernel(u, v, u_edge_indices, u_edge_values, v_edge_indices, v_edge_values, interaction_pair, label):
    m, d = u.shape
    n = v.shape[0]
    grid = (m // _BM, n // _BN)
    out = pl.pallas_call(
        _matmul_sigmoid_kernel,
        grid=grid,
        in_specs=[
            pl.BlockSpec((_BM, d), lambda i, j: (i, 0)),
            pl.BlockSpec((_BN, d), lambda i, j: (j, 0)),
        ],
        out_specs=pl.BlockSpec((_BM, _BN), lambda i, j: (i, j)),
        out_shape=jax.ShapeDtypeStruct((m, n), jnp.float32),
    )(u, v)
    return (out, u, v)
